# baseline (speedup 1.0000x reference)
import jax
import jax.numpy as jnp
from jax.experimental import pallas as pl

_B, _N, _IN_DIM, _FEA, _H = 8, 64, 512, 256, 8
_NB = _B * _N
_NEG_SLOPE = 0.2


def _gat_proto_kernel(x_ref, wlin_ref, blin_ref, wgat_ref, asrc_ref,
                      adst_ref, bias_ref, out_ref):
    cat_t = jnp.concatenate([asrc_ref[...], adst_ref[...]], axis=0).T
    tiled = jnp.concatenate([cat_t] * _H, axis=0)
    row_head = jax.lax.broadcasted_iota(
        jnp.int32, (_H * _FEA, 2 * _H), 0) // _FEA
    col_head = jax.lax.broadcasted_iota(
        jnp.int32, (_H * _FEA, 2 * _H), 1) % _H
    bd = jnp.where(row_head == col_head, tiled, 0.0)
    vsd = jnp.dot(wgat_ref[...], bd, preferred_element_type=jnp.float32)

    h = jnp.dot(x_ref[...], wlin_ref[...],
                preferred_element_type=jnp.float32) + blin_ref[...]
    aa = jnp.dot(h, vsd, preferred_element_type=jnp.float32)
    aa_t = aa.T

    lane = jax.lax.broadcasted_iota(jnp.int32, (1, 2 * _N), 1)
    left = lane < _N

    w_rows_list = []
    for b in range(_B):
        sl = slice(b * _N, (b + 1) * _N)
        w_pairs = []
        for p in range(_H // 2):
            h0, h1 = 2 * p, 2 * p + 1
            asr = jnp.concatenate(
                [aa_t[h0:h0 + 1, sl], aa_t[h1:h1 + 1, sl]], axis=1)
            ad0 = aa[sl, _H + h0:_H + h0 + 1]
            ad1 = aa[sl, _H + h1:_H + h1 + 1]
            s = jnp.where(left, ad0, ad1) + asr
            s = jnp.where(s >= 0, s, _NEG_SLOPE * s)
            m = jnp.max(s, axis=1, keepdims=True)
            ex = jnp.exp(s - m)
            d0 = jnp.sum(ex[:, :_N], axis=1, keepdims=True)
            d1 = jnp.sum(ex[:, _N:], axis=1, keepdims=True)
            inv2 = jnp.where(left, 1.0 / (d0 + 1e-16), 1.0 / (d1 + 1e-16))
            w_pairs.append(jnp.sum(ex * inv2, axis=0, keepdims=True))
        w_pack = jnp.concatenate(w_pairs, axis=0)
        w_rows_list.append(jnp.concatenate(
            [w_pack[:, :_N], w_pack[:, _N:]], axis=0))

    w_rows = jnp.concatenate(w_rows_list, axis=0)
    tiled_w = jnp.concatenate([w_rows] * _B, axis=1)
    row_b = jax.lax.broadcasted_iota(jnp.int32, (_B * _H, _NB), 0) // _H
    col_b = jax.lax.broadcasted_iota(jnp.int32, (_B * _H, _NB), 1) // _N
    a_mat = jnp.where(row_b == col_b, tiled_w, 0.0)
    g = jnp.dot(a_mat, h, preferred_element_type=jnp.float32)
    g2 = g.reshape(_B, _H * _FEA)
    order = (0, 2, 4, 6, 1, 3, 5, 7)
    acc = jnp.zeros((_B, _FEA), dtype=jnp.float32)
    for k in range(_H):
        hh = order[k]
        acc = acc + jnp.dot(g2[:, k * _FEA:(k + 1) * _FEA],
                            wgat_ref[:, hh * _FEA:(hh + 1) * _FEA],
                            preferred_element_type=jnp.float32)
    out_ref[...] = acc * (1.0 / (_N * _H)) + bias_ref[...]


def kernel(x, W_lin, b_lin, W_gat, att_src, att_dst, bias_gat):
    x2 = x.reshape(_NB, _IN_DIM)
    blin2 = b_lin.reshape(1, _FEA)
    bias2 = bias_gat.reshape(1, _FEA)
    out = pl.pallas_call(
        _gat_proto_kernel,
        in_specs=[
            pl.BlockSpec((_NB, _IN_DIM), lambda: (0, 0)),
            pl.BlockSpec((_IN_DIM, _FEA), lambda: (0, 0)),
            pl.BlockSpec((1, _FEA), lambda: (0, 0)),
            pl.BlockSpec((_FEA, _H * _FEA), lambda: (0, 0)),
            pl.BlockSpec((_H, _FEA), lambda: (0, 0)),
            pl.BlockSpec((_H, _FEA), lambda: (0, 0)),
            pl.BlockSpec((1, _FEA), lambda: (0, 0)),
        ],
        out_specs=pl.BlockSpec((_B, _FEA), lambda: (0, 0)),
        out_shape=jax.ShapeDtypeStruct((_B, _FEA), jnp.float32),
    )(x2, W_lin, blin2, W_gat, att_src, att_dst, bias2)
    return out


# speedup vs baseline: 1.0033x; 1.0033x over previous
import jax
import jax.numpy as jnp
from jax.experimental import pallas as pl

_B, _N, _IN_DIM, _FEA, _H = 8, 64, 512, 256, 8
_NB = _B * _N
_NEG_SLOPE = 0.2


def _gat_proto_kernel(x_ref, wlin_ref, blin_ref, wgat_ref, wgstack_ref,
                      asrc_ref, adst_ref, bias_ref, out_ref):
    cat_t = jnp.concatenate([asrc_ref[...], adst_ref[...]], axis=0).T
    tiled = jnp.concatenate([cat_t] * _H, axis=0)
    row_head = jax.lax.broadcasted_iota(
        jnp.int32, (_H * _FEA, 2 * _H), 0) // _FEA
    col_head = jax.lax.broadcasted_iota(
        jnp.int32, (_H * _FEA, 2 * _H), 1) % _H
    bd = jnp.where(row_head == col_head, tiled, 0.0)
    vsd = jnp.dot(wgat_ref[...], bd, preferred_element_type=jnp.float32)

    h = jnp.dot(x_ref[...], wlin_ref[...],
                preferred_element_type=jnp.float32) + blin_ref[...]
    aa = jnp.dot(h, vsd, preferred_element_type=jnp.float32)

    qrow = jax.lax.broadcasted_iota(jnp.int32, (_H, _NB), 0)
    qcol = jax.lax.broadcasted_iota(jnp.int32, (_H, _NB), 1) // _N
    qm = jnp.where(qrow == qcol, 1.0, 0.0)
    prow = jax.lax.broadcasted_iota(jnp.int32, (_NB, _H), 0) // _N
    pcol = jax.lax.broadcasted_iota(jnp.int32, (_NB, _H), 1)
    pm = jnp.where(prow == pcol, 1.0, 0.0)

    xsrc = jnp.dot(aa[:, :_H], qm, preferred_element_type=jnp.float32)
    dmat = aa[:, _H:].reshape(_B, _N, _H).transpose(0, 2, 1).reshape(_B, _NB)
    dstf = jnp.dot(pm, dmat, preferred_element_type=jnp.float32)
    s = xsrc + dstf
    s = jnp.where(s >= 0, s, _NEG_SLOPE * s)

    m = jnp.max(s.reshape(_B, _N, _NB), axis=1)
    ex = jnp.exp(s - jnp.dot(pm, m, preferred_element_type=jnp.float32))
    den = jnp.dot(qm, ex, preferred_element_type=jnp.float32)
    inv = 1.0 / (den + 1e-16)
    exs = ex * jnp.dot(pm, inv, preferred_element_type=jnp.float32)
    w2 = jnp.dot(exs, pm, preferred_element_type=jnp.float32)

    w2t = w2.T
    tiled_w = jnp.concatenate([w2t] * _B, axis=0)
    row_b = jax.lax.broadcasted_iota(jnp.int32, (_B * _H, _NB), 0) // _H
    col_b = jax.lax.broadcasted_iota(jnp.int32, (_B * _H, _NB), 1) // _N
    a_mat = jnp.where(row_b == col_b, tiled_w, 0.0)
    g = jnp.dot(a_mat, h, preferred_element_type=jnp.float32)
    g2 = g.reshape(_B, _H * _FEA)
    out = jnp.dot(g2, wgstack_ref[...], preferred_element_type=jnp.float32)
    out_ref[...] = out * (1.0 / (_N * _H)) + bias_ref[...]


def kernel(x, W_lin, b_lin, W_gat, att_src, att_dst, bias_gat):
    x2 = x.reshape(_NB, _IN_DIM)
    blin2 = b_lin.reshape(1, _FEA)
    bias2 = bias_gat.reshape(1, _FEA)
    wg_stack = W_gat.reshape(_FEA, _H, _FEA).transpose(1, 0, 2).reshape(
        _H * _FEA, _FEA)
    out = pl.pallas_call(
        _gat_proto_kernel,
        in_specs=[
            pl.BlockSpec((_NB, _IN_DIM), lambda: (0, 0)),
            pl.BlockSpec((_IN_DIM, _FEA), lambda: (0, 0)),
            pl.BlockSpec((1, _FEA), lambda: (0, 0)),
            pl.BlockSpec((_FEA, _H * _FEA), lambda: (0, 0)),
            pl.BlockSpec((_H * _FEA, _FEA), lambda: (0, 0)),
            pl.BlockSpec((_H, _FEA), lambda: (0, 0)),
            pl.BlockSpec((_H, _FEA), lambda: (0, 0)),
            pl.BlockSpec((1, _FEA), lambda: (0, 0)),
        ],
        out_specs=pl.BlockSpec((_B, _FEA), lambda: (0, 0)),
        out_shape=jax.ShapeDtypeStruct((_B, _FEA), jnp.float32),
    )(x2, W_lin, blin2, W_gat, wg_stack, att_src, att_dst, bias2)
    return out


# speedup vs baseline: 1.6333x; 1.6279x over previous
import jax
import jax.numpy as jnp
from jax.experimental import pallas as pl

_B, _N, _IN_DIM, _FEA, _H = 8, 64, 512, 256, 8
_NB = _B * _N
_NEG_SLOPE = 0.2


def _gat_proto_kernel(x_ref, wlin_ref, blin_ref, wgat_ref,
                      asrc_ref, adst_ref, bias_ref, out_ref):
    cat_t = jnp.concatenate([asrc_ref[...], adst_ref[...]], axis=0).T
    tiled = jnp.concatenate([cat_t] * _H, axis=0)
    row_head = jax.lax.broadcasted_iota(
        jnp.int32, (_H * _FEA, 2 * _H), 0) // _FEA
    col_head = jax.lax.broadcasted_iota(
        jnp.int32, (_H * _FEA, 2 * _H), 1) % _H
    bd = jnp.where(row_head == col_head, tiled, 0.0)
    vsd = jnp.dot(wgat_ref[...], bd, preferred_element_type=jnp.float32)

    h = jnp.dot(x_ref[...], wlin_ref[...],
                preferred_element_type=jnp.float32) + blin_ref[...]
    aa = jnp.dot(h, vsd, preferred_element_type=jnp.float32)

    qrow = jax.lax.broadcasted_iota(jnp.int32, (_H, _NB), 0)
    qcol = jax.lax.broadcasted_iota(jnp.int32, (_H, _NB), 1) // _N
    qm = jnp.where(qrow == qcol, 1.0, 0.0)
    prow = jax.lax.broadcasted_iota(jnp.int32, (_NB, _H), 0) // _N
    pcol = jax.lax.broadcasted_iota(jnp.int32, (_NB, _H), 1)
    pm = jnp.where(prow == pcol, 1.0, 0.0)

    xsrc = jnp.dot(aa[:, :_H], qm, preferred_element_type=jnp.float32)
    dmat = aa[:, _H:].reshape(_B, _N, _H).transpose(0, 2, 1).reshape(_B, _NB)
    dstf = jnp.dot(pm, dmat, preferred_element_type=jnp.float32)
    s = xsrc + dstf
    s = jnp.where(s >= 0, s, _NEG_SLOPE * s)

    m = jnp.max(s.reshape(_B, _N, _NB), axis=1)
    ex = jnp.exp(s - jnp.dot(pm, m, preferred_element_type=jnp.float32))
    den = jnp.dot(qm, ex, preferred_element_type=jnp.float32)
    inv = 1.0 / (den + 1e-16)
    exs = ex * jnp.dot(pm, inv, preferred_element_type=jnp.float32)
    w2 = jnp.dot(exs, pm, preferred_element_type=jnp.float32)

    w2t = w2.T
    tiled_w = jnp.concatenate([w2t] * _B, axis=0)
    row_b = jax.lax.broadcasted_iota(jnp.int32, (_B * _H, _NB), 0) // _H
    col_b = jax.lax.broadcasted_iota(jnp.int32, (_B * _H, _NB), 1) // _N
    a_mat = jnp.where(row_b == col_b, tiled_w, 0.0)
    g = jnp.dot(a_mat, h, preferred_element_type=jnp.float32)
    g2 = g.reshape(_B, _H * _FEA)
    acc = jnp.zeros((_B, _FEA), dtype=jnp.float32)
    for k in range(_H):
        acc = acc + jnp.dot(g2[:, k * _FEA:(k + 1) * _FEA],
                            wgat_ref[:, k * _FEA:(k + 1) * _FEA],
                            preferred_element_type=jnp.float32)
    out_ref[...] = acc * (1.0 / (_N * _H)) + bias_ref[...]


def kernel(x, W_lin, b_lin, W_gat, att_src, att_dst, bias_gat):
    x2 = x.reshape(_NB, _IN_DIM)
    blin2 = b_lin.reshape(1, _FEA)
    bias2 = bias_gat.reshape(1, _FEA)
    out = pl.pallas_call(
        _gat_proto_kernel,
        in_specs=[
            pl.BlockSpec((_NB, _IN_DIM), lambda: (0, 0)),
            pl.BlockSpec((_IN_DIM, _FEA), lambda: (0, 0)),
            pl.BlockSpec((1, _FEA), lambda: (0, 0)),
            pl.BlockSpec((_FEA, _H * _FEA), lambda: (0, 0)),
            pl.BlockSpec((_H, _FEA), lambda: (0, 0)),
            pl.BlockSpec((_H, _FEA), lambda: (0, 0)),
            pl.BlockSpec((1, _FEA), lambda: (0, 0)),
        ],
        out_specs=pl.BlockSpec((_B, _FEA), lambda: (0, 0)),
        out_shape=jax.ShapeDtypeStruct((_B, _FEA), jnp.float32),
    )(x2, W_lin, blin2, W_gat, att_src, att_dst, bias2)
    return out
